# Initial kernel scaffold; baseline (speedup 1.0000x reference)
import jax
import jax.numpy as jnp
from jax.experimental import pallas as pl


def kernel(x, edge_index, i, W1, W2, b1, W3, b3):
    raise NotImplementedError("write your pallas kernel here")


# speedup vs baseline: 11.6686x; 11.6686x over previous
import functools

import jax
import jax.numpy as jnp
from jax import lax
from jax.experimental import pallas as pl
from jax.experimental.pallas import tpu as pltpu
from jax.experimental.pallas import tpu_sc as plsc

N = 10000
E = 320000
F = 128
H = 128
L = 64

NC = 2
NS = 16
NW = NC * NS
EPW = E // NW
CHUNK = 80
NCHUNK = EPW // CHUNK
RPT = 624
TAIL_OFF = RPT * NS
TAIL = N - TAIL_OFF
DCOL = 64

_mesh = plsc.VectorSubcoreMesh(core_axis_name="c", subcore_axis_name="s")


@functools.partial(
    pl.kernel,
    out_type=jax.ShapeDtypeStruct((NC, N, H), jnp.float32),
    mesh=_mesh,
    scratch_types=[
        pltpu.VMEM((CHUNK,), jnp.int32),
        pltpu.VMEM((CHUNK,), jnp.int32),
        pltpu.VMEM((CHUNK, H), jnp.float32),
        pltpu.VMEM((CHUNK, H), jnp.float32),
        pltpu.VMEM_SHARED((N, H), jnp.float32),
    ],
)
def _deg_kernel(src_hbm, dst_hbm, zeros_hbm, onesa_hbm, onesb_hbm, out_hbm,
                sidx, didx, onesa_v, onesb_v, hist):
    cid = lax.axis_index("c")
    sid = lax.axis_index("s")
    wid = cid * NS + sid

    pltpu.sync_copy(onesa_hbm, onesa_v)
    pltpu.sync_copy(onesb_hbm, onesb_v)

    pltpu.sync_copy(zeros_hbm.at[pl.ds(sid * RPT, RPT)], hist.at[pl.ds(sid * RPT, RPT)])

    @pl.when(sid == 0)
    def _():
        pltpu.sync_copy(zeros_hbm.at[pl.ds(TAIL_OFF, TAIL)], hist.at[pl.ds(TAIL_OFF, TAIL)])

    plsc.subcore_barrier()

    base = wid * EPW

    def body(g, carry):
        off = base + g * CHUNK
        pltpu.sync_copy(src_hbm.at[pl.ds(off, CHUNK)], sidx)
        pltpu.sync_copy(dst_hbm.at[pl.ds(off, CHUNK)], didx)
        pltpu.sync_copy(onesa_v, hist.at[sidx], add=True)
        pltpu.sync_copy(onesb_v, hist.at[didx], add=True)
        return carry

    lax.fori_loop(0, NCHUNK, body, 0)
    plsc.subcore_barrier()

    pltpu.sync_copy(hist.at[pl.ds(sid * RPT, RPT)], out_hbm.at[cid, pl.ds(sid * RPT, RPT)])

    @pl.when(sid == 0)
    def _():
        pltpu.sync_copy(hist.at[pl.ds(TAIL_OFF, TAIL)], out_hbm.at[cid, pl.ds(TAIL_OFF, TAIL)])


@functools.partial(
    pl.kernel,
    out_type=jax.ShapeDtypeStruct((NC, N, H), jnp.float32),
    mesh=_mesh,
    scratch_types=[
        pltpu.VMEM((CHUNK,), jnp.int32),
        pltpu.VMEM((CHUNK,), jnp.int32),
        pltpu.VMEM((CHUNK, H), jnp.float32),
        pltpu.VMEM_SHARED((N, H), jnp.float32),
        pltpu.SemaphoreType.DMA,
    ],
)
def _edge_kernel(z_hbm, src_hbm, dst_hbm, zrows_hbm, out_hbm, sidx, didx, rows, acc, sem):
    cid = lax.axis_index("c")
    sid = lax.axis_index("s")
    wid = cid * NS + sid

    pltpu.sync_copy(zrows_hbm.at[pl.ds(sid * RPT, RPT)], acc.at[pl.ds(sid * RPT, RPT)])

    @pl.when(sid == 0)
    def _():
        pltpu.sync_copy(zrows_hbm.at[pl.ds(TAIL_OFF, TAIL)], acc.at[pl.ds(TAIL_OFF, TAIL)])

    plsc.subcore_barrier()

    base = wid * EPW

    def body(g, carry):
        off = base + g * CHUNK
        pltpu.sync_copy(src_hbm.at[pl.ds(off, CHUNK)], sidx)
        pltpu.sync_copy(dst_hbm.at[pl.ds(off, CHUNK)], didx)
        pltpu.async_copy(z_hbm.at[sidx], rows, sem).wait()
        pltpu.sync_copy(rows, acc.at[didx], add=True)
        return carry

    lax.fori_loop(0, NCHUNK, body, 0)
    plsc.subcore_barrier()

    pltpu.sync_copy(acc.at[pl.ds(sid * RPT, RPT)], out_hbm.at[cid, pl.ds(sid * RPT, RPT)])

    @pl.when(sid == 0)
    def _():
        pltpu.sync_copy(acc.at[pl.ds(TAIL_OFF, TAIL)], out_hbm.at[cid, pl.ds(TAIL_OFF, TAIL)])


BR = 2000


def _z_body(x_ref, w1_ref, degp_ref, z_ref):
    do = degp_ref[0, :, 0:1] + degp_ref[1, :, 0:1]
    iso = lax.rsqrt(jnp.maximum(do, 1.0))
    y = jnp.dot(x_ref[...], w1_ref[...], preferred_element_type=jnp.float32)
    z_ref[...] = y * iso


def _z_call(x, W1, degp):
    return pl.pallas_call(
        _z_body,
        grid=(N // BR,),
        in_specs=[
            pl.BlockSpec((BR, F), lambda i: (i, 0)),
            pl.BlockSpec((F, H), lambda i: (0, 0)),
            pl.BlockSpec((NC, BR, H), lambda i: (0, i, 0)),
        ],
        out_specs=pl.BlockSpec((BR, H), lambda i: (i, 0)),
        out_shape=jax.ShapeDtypeStruct((N, H), jnp.float32),
    )(x, W1, degp)


def _head_body(p_ref, degp_ref, x_ref, w2_ref, b1_ref, w3_ref, b3_ref, o_ref):
    agg = p_ref[0] + p_ref[1]
    di = degp_ref[0, :, DCOL:DCOL + 1] + degp_ref[1, :, DCOL:DCOL + 1]
    isi = lax.rsqrt(jnp.maximum(di, 1.0))
    h = agg * isi
    h = h + jnp.dot(x_ref[...], w2_ref[...], preferred_element_type=jnp.float32)
    h = h + b1_ref[...]
    h = jnp.where(h > 0, h, 0.2 * h)
    logits = jnp.dot(h, w3_ref[...], preferred_element_type=jnp.float32) + b3_ref[...]
    m = jnp.max(logits, axis=1, keepdims=True)
    ex = jnp.exp(logits - m)
    o_ref[...] = ex / jnp.sum(ex, axis=1, keepdims=True)


def _head_call(p, degp, x, W2, b1, W3, b3):
    return pl.pallas_call(
        _head_body,
        grid=(N // BR,),
        in_specs=[
            pl.BlockSpec((NC, BR, H), lambda i: (0, i, 0)),
            pl.BlockSpec((NC, BR, H), lambda i: (0, i, 0)),
            pl.BlockSpec((BR, F), lambda i: (i, 0)),
            pl.BlockSpec((F, H), lambda i: (0, 0)),
            pl.BlockSpec((1, H), lambda i: (0, 0)),
            pl.BlockSpec((H, L), lambda i: (0, 0)),
            pl.BlockSpec((1, L), lambda i: (0, 0)),
        ],
        out_specs=pl.BlockSpec((BR, L), lambda i: (i, 0)),
        out_shape=jax.ShapeDtypeStruct((N, L), jnp.float32),
    )(p, degp, x, W2, b1, W3, b3)


def kernel(x, edge_index, i, W1, W2, b1, W3, b3):
    del i
    src = jnp.asarray(edge_index[0], jnp.int32)
    dst = jnp.asarray(edge_index[1], jnp.int32)
    zeros_rows = jnp.zeros((N, H), jnp.float32)
    col = jnp.arange(H)
    onesa = jnp.broadcast_to((col == 0).astype(jnp.float32), (CHUNK, H))
    onesb = jnp.broadcast_to((col == DCOL).astype(jnp.float32), (CHUNK, H))

    degp = _deg_kernel(src, dst, zeros_rows, onesa, onesb)
    z = _z_call(x, W1, degp)
    p = _edge_kernel(z, src, dst, zeros_rows)
    out = _head_call(p, degp, x, W2,
                     b1.reshape(1, H), W3, b3.reshape(1, L))
    return out


# speedup vs baseline: 16.9910x; 1.4561x over previous
import functools

import jax
import jax.numpy as jnp
from jax import lax
from jax.experimental import pallas as pl
from jax.experimental.pallas import tpu as pltpu
from jax.experimental.pallas import tpu_sc as plsc

N = 10000
E = 320000
F = 128
H = 128
L = 64

NC = 2
NS = 16
NW = NC * NS
EPW = E // NW
CHUNK = 80
NCHUNK = EPW // CHUNK
RPT = 624
TAIL_OFF = RPT * NS
TAIL = N - TAIL_OFF
DCOL = 64

_mesh = plsc.VectorSubcoreMesh(core_axis_name="c", subcore_axis_name="s")


@functools.partial(
    pl.kernel,
    out_type=jax.ShapeDtypeStruct((NC, N, H), jnp.float32),
    mesh=_mesh,
    scratch_types=[
        pltpu.VMEM((CHUNK,), jnp.int32),
        pltpu.VMEM((CHUNK,), jnp.int32),
        pltpu.VMEM((CHUNK, H), jnp.float32),
        pltpu.VMEM((CHUNK, H), jnp.float32),
        pltpu.VMEM_SHARED((N, H), jnp.float32),
    ],
)
def _deg_kernel(src_hbm, dst_hbm, zeros_hbm, onesa_hbm, onesb_hbm, out_hbm,
                sidx, didx, onesa_v, onesb_v, hist):
    cid = lax.axis_index("c")
    sid = lax.axis_index("s")
    wid = cid * NS + sid

    pltpu.sync_copy(onesa_hbm, onesa_v)
    pltpu.sync_copy(onesb_hbm, onesb_v)

    pltpu.sync_copy(zeros_hbm.at[pl.ds(sid * RPT, RPT)], hist.at[pl.ds(sid * RPT, RPT)])

    @pl.when(sid == 0)
    def _():
        pltpu.sync_copy(zeros_hbm.at[pl.ds(TAIL_OFF, TAIL)], hist.at[pl.ds(TAIL_OFF, TAIL)])

    plsc.subcore_barrier()

    base = wid * EPW

    def body(g, carry):
        off = base + g * CHUNK
        pltpu.sync_copy(src_hbm.at[pl.ds(off, CHUNK)], sidx)
        pltpu.sync_copy(dst_hbm.at[pl.ds(off, CHUNK)], didx)
        pltpu.sync_copy(onesa_v, hist.at[sidx], add=True)
        pltpu.sync_copy(onesb_v, hist.at[didx], add=True)
        return carry

    lax.fori_loop(0, NCHUNK, body, 0)
    plsc.subcore_barrier()

    pltpu.sync_copy(hist.at[pl.ds(sid * RPT, RPT)], out_hbm.at[cid, pl.ds(sid * RPT, RPT)])

    @pl.when(sid == 0)
    def _():
        pltpu.sync_copy(hist.at[pl.ds(TAIL_OFF, TAIL)], out_hbm.at[cid, pl.ds(TAIL_OFF, TAIL)])


CCH = 80
CNCH = EPW // CCH


@functools.partial(
    pl.kernel,
    out_type=jax.ShapeDtypeStruct((NC, N, H), jnp.float32),
    mesh=_mesh,
    scratch_types=[
        pltpu.VMEM((EPW,), jnp.int32),
        pltpu.VMEM((CNCH, CCH), jnp.int32),
        pltpu.VMEM((CCH, H), jnp.float32),
        pltpu.VMEM((CCH, H), jnp.float32),
        pltpu.VMEM_SHARED((N, H), jnp.float32),
        pltpu.SemaphoreType.DMA,
        pltpu.SemaphoreType.DMA,
    ],
)
def _edge_kernel(z_hbm, src_hbm, dst3_hbm, zrows_hbm, out_hbm,
                 sidx, didx_all, rows0, rows1, acc, sem0, sem1):
    cid = lax.axis_index("c")
    sid = lax.axis_index("s")
    wid = cid * NS + sid

    pltpu.sync_copy(src_hbm.at[pl.ds(wid * EPW, EPW)], sidx)
    pltpu.sync_copy(dst3_hbm.at[wid], didx_all)

    pltpu.sync_copy(zrows_hbm.at[pl.ds(sid * RPT, RPT)], acc.at[pl.ds(sid * RPT, RPT)])

    @pl.when(sid == 0)
    def _():
        pltpu.sync_copy(zrows_hbm.at[pl.ds(TAIL_OFF, TAIL)], acc.at[pl.ds(TAIL_OFF, TAIL)])

    plsc.subcore_barrier()

    dummy0 = pltpu.make_async_copy(zrows_hbm.at[pl.ds(0, CCH)], rows0, sem0)
    dummy1 = pltpu.make_async_copy(zrows_hbm.at[pl.ds(0, CCH)], rows1, sem1)

    pltpu.async_copy(z_hbm.at[sidx.at[pl.ds(0, CCH)]], rows0, sem0)

    def body(g, carry):
        c0 = 2 * g
        c1 = 2 * g + 1
        pltpu.async_copy(z_hbm.at[sidx.at[pl.ds(c1 * CCH, CCH)]], rows1, sem1)
        dummy0.wait()
        pltpu.sync_copy(rows0, acc.at[didx_all.at[c0]], add=True)
        pltpu.async_copy(z_hbm.at[sidx.at[pl.ds((c0 + 2) * CCH, CCH)]], rows0, sem0)
        dummy1.wait()
        pltpu.sync_copy(rows1, acc.at[didx_all.at[c1]], add=True)
        return carry

    lax.fori_loop(0, CNCH // 2, body, 0)
    dummy0.wait()
    pltpu.sync_copy(rows0, acc.at[didx_all.at[CNCH - 1]], add=True)
    plsc.subcore_barrier()

    pltpu.sync_copy(acc.at[pl.ds(sid * RPT, RPT)], out_hbm.at[cid, pl.ds(sid * RPT, RPT)])

    @pl.when(sid == 0)
    def _():
        pltpu.sync_copy(acc.at[pl.ds(TAIL_OFF, TAIL)], out_hbm.at[cid, pl.ds(TAIL_OFF, TAIL)])


BR = 2000


def _z_body(x_ref, w1_ref, degp_ref, z_ref):
    do = degp_ref[0, :, 0:1] + degp_ref[1, :, 0:1]
    iso = lax.rsqrt(jnp.maximum(do, 1.0))
    y = jnp.dot(x_ref[...], w1_ref[...], preferred_element_type=jnp.float32)
    z_ref[...] = y * iso


def _z_call(x, W1, degp):
    return pl.pallas_call(
        _z_body,
        grid=(N // BR,),
        in_specs=[
            pl.BlockSpec((BR, F), lambda i: (i, 0)),
            pl.BlockSpec((F, H), lambda i: (0, 0)),
            pl.BlockSpec((NC, BR, H), lambda i: (0, i, 0)),
        ],
        out_specs=pl.BlockSpec((BR, H), lambda i: (i, 0)),
        out_shape=jax.ShapeDtypeStruct((N, H), jnp.float32),
    )(x, W1, degp)


def _head_body(p_ref, degp_ref, x_ref, w2_ref, b1_ref, w3_ref, b3_ref, o_ref):
    agg = p_ref[0] + p_ref[1]
    di = degp_ref[0, :, DCOL:DCOL + 1] + degp_ref[1, :, DCOL:DCOL + 1]
    isi = lax.rsqrt(jnp.maximum(di, 1.0))
    h = agg * isi
    h = h + jnp.dot(x_ref[...], w2_ref[...], preferred_element_type=jnp.float32)
    h = h + b1_ref[...]
    h = jnp.where(h > 0, h, 0.2 * h)
    logits = jnp.dot(h, w3_ref[...], preferred_element_type=jnp.float32) + b3_ref[...]
    m = jnp.max(logits, axis=1, keepdims=True)
    ex = jnp.exp(logits - m)
    o_ref[...] = ex / jnp.sum(ex, axis=1, keepdims=True)


def _head_call(p, degp, x, W2, b1, W3, b3):
    return pl.pallas_call(
        _head_body,
        grid=(N // BR,),
        in_specs=[
            pl.BlockSpec((NC, BR, H), lambda i: (0, i, 0)),
            pl.BlockSpec((NC, BR, H), lambda i: (0, i, 0)),
            pl.BlockSpec((BR, F), lambda i: (i, 0)),
            pl.BlockSpec((F, H), lambda i: (0, 0)),
            pl.BlockSpec((1, H), lambda i: (0, 0)),
            pl.BlockSpec((H, L), lambda i: (0, 0)),
            pl.BlockSpec((1, L), lambda i: (0, 0)),
        ],
        out_specs=pl.BlockSpec((BR, L), lambda i: (i, 0)),
        out_shape=jax.ShapeDtypeStruct((N, L), jnp.float32),
    )(p, degp, x, W2, b1, W3, b3)


def kernel(x, edge_index, i, W1, W2, b1, W3, b3):
    del i
    src = jnp.asarray(edge_index[0], jnp.int32)
    dst = jnp.asarray(edge_index[1], jnp.int32)
    zeros_rows = jnp.zeros((N, H), jnp.float32)
    col = jnp.arange(H)
    onesa = jnp.broadcast_to((col == 0).astype(jnp.float32), (CHUNK, H))
    onesb = jnp.broadcast_to((col == DCOL).astype(jnp.float32), (CHUNK, H))
    dst3 = dst.reshape(NW, CNCH, CCH)

    degp = _deg_kernel(src, dst, zeros_rows, onesa, onesb)
    z = _z_call(x, W1, degp)
    p = _edge_kernel(z, src, dst3, zeros_rows)
    out = _head_call(p, degp, x, W2,
                     b1.reshape(1, H), W3, b3.reshape(1, L))
    return out


# speedup vs baseline: 31.9450x; 1.8801x over previous
import functools

import jax
import jax.numpy as jnp
from jax import lax
from jax.experimental import pallas as pl
from jax.experimental.pallas import tpu as pltpu
from jax.experimental.pallas import tpu_sc as plsc

N = 10000
E = 320000
F = 128
H = 128
L = 64

NC = 2
NS = 16
NW = NC * NS
EPW = E // NW
CHUNK = 80
NCHUNK = EPW // CHUNK
RPT = 624
TAIL_OFF = RPT * NS
TAIL = N - TAIL_OFF
NP = 10240

_mesh = plsc.VectorSubcoreMesh(core_axis_name="c", subcore_axis_name="s")


@functools.partial(
    pl.kernel,
    out_type=jax.ShapeDtypeStruct((2 * NC, 1, NP), jnp.float32),
    mesh=_mesh,
    scratch_types=[
        pltpu.VMEM((NCHUNK, CHUNK), jnp.int32),
        pltpu.VMEM((NCHUNK, CHUNK), jnp.int32),
        pltpu.VMEM((CHUNK,), jnp.float32),
        pltpu.VMEM_SHARED((NP,), jnp.float32),
        pltpu.VMEM_SHARED((NP,), jnp.float32),
    ],
)
def _deg_kernel(src3_hbm, dst3_hbm, zeros1_hbm, ones1_hbm, out_hbm,
                sidx_all, didx_all, ones_v, hs, hd):
    cid = lax.axis_index("c")
    sid = lax.axis_index("s")
    wid = cid * NS + sid

    pltpu.sync_copy(src3_hbm.at[wid], sidx_all)
    pltpu.sync_copy(dst3_hbm.at[wid], didx_all)
    pltpu.sync_copy(ones1_hbm, ones_v)

    @pl.when(sid == 0)
    def _():
        pltpu.sync_copy(zeros1_hbm, hs)
        pltpu.sync_copy(zeros1_hbm, hd)

    plsc.subcore_barrier()

    def body(g, carry):
        pltpu.sync_copy(ones_v, hs.at[sidx_all.at[g]], add=True)
        pltpu.sync_copy(ones_v, hd.at[didx_all.at[g]], add=True)
        return carry

    lax.fori_loop(0, NCHUNK, body, 0)
    plsc.subcore_barrier()

    pltpu.sync_copy(hs.at[pl.ds(sid * 640, 640)], out_hbm.at[cid * 2, 0, pl.ds(sid * 640, 640)])
    pltpu.sync_copy(hd.at[pl.ds(sid * 640, 640)], out_hbm.at[cid * 2 + 1, 0, pl.ds(sid * 640, 640)])


def _invs_body(degp_ref, o_ref):
    d4 = degp_ref[...][:, 0, :]
    rows = lax.broadcasted_iota(jnp.int32, (2 * NC, 1), 0)
    sel_o = (rows % 2 == 0).astype(jnp.float32)
    sel_i = 1.0 - sel_o
    cd = (((0,), (0,)), ((), ()))
    do = lax.dot_general(d4, sel_o, cd, preferred_element_type=jnp.float32)
    di = lax.dot_general(d4, sel_i, cd, preferred_element_type=jnp.float32)
    o_ref[0] = lax.rsqrt(jnp.maximum(do, 1.0))
    o_ref[1] = lax.rsqrt(jnp.maximum(di, 1.0))


def _invs_call(degp):
    return pl.pallas_call(
        _invs_body,
        out_shape=jax.ShapeDtypeStruct((2, NP, 1), jnp.float32),
    )(degp)


CCH = 80
CNCH = EPW // CCH


@functools.partial(
    pl.kernel,
    out_type=jax.ShapeDtypeStruct((NC, N, H), jnp.float32),
    mesh=_mesh,
    scratch_types=[
        pltpu.VMEM((EPW,), jnp.int32),
        pltpu.VMEM((CNCH, CCH), jnp.int32),
        pltpu.VMEM((CCH, H), jnp.float32),
        pltpu.VMEM((CCH, H), jnp.float32),
        pltpu.VMEM_SHARED((N, H), jnp.float32),
        pltpu.SemaphoreType.DMA,
        pltpu.SemaphoreType.DMA,
    ],
)
def _edge_kernel(z_hbm, src_hbm, dst3_hbm, zrows_hbm, out_hbm,
                 sidx, didx_all, rows0, rows1, acc, sem0, sem1):
    cid = lax.axis_index("c")
    sid = lax.axis_index("s")
    wid = cid * NS + sid

    pltpu.sync_copy(src_hbm.at[pl.ds(wid * EPW, EPW)], sidx)
    pltpu.sync_copy(dst3_hbm.at[wid], didx_all)

    pltpu.sync_copy(zrows_hbm.at[pl.ds(sid * RPT, RPT)], acc.at[pl.ds(sid * RPT, RPT)])

    @pl.when(sid == 0)
    def _():
        pltpu.sync_copy(zrows_hbm.at[pl.ds(TAIL_OFF, TAIL)], acc.at[pl.ds(TAIL_OFF, TAIL)])

    plsc.subcore_barrier()

    dummy0 = pltpu.make_async_copy(zrows_hbm.at[pl.ds(0, CCH)], rows0, sem0)
    dummy1 = pltpu.make_async_copy(zrows_hbm.at[pl.ds(0, CCH)], rows1, sem1)

    pltpu.async_copy(z_hbm.at[sidx.at[pl.ds(0, CCH)]], rows0, sem0)

    def body(g, carry):
        c0 = 2 * g
        c1 = 2 * g + 1
        pltpu.async_copy(z_hbm.at[sidx.at[pl.ds(c1 * CCH, CCH)]], rows1, sem1)
        dummy0.wait()
        pltpu.sync_copy(rows0, acc.at[didx_all.at[c0]], add=True)
        pltpu.async_copy(z_hbm.at[sidx.at[pl.ds((c0 + 2) * CCH, CCH)]], rows0, sem0)
        dummy1.wait()
        pltpu.sync_copy(rows1, acc.at[didx_all.at[c1]], add=True)
        return carry

    lax.fori_loop(0, CNCH // 2, body, 0)
    dummy0.wait()
    pltpu.sync_copy(rows0, acc.at[didx_all.at[CNCH - 1]], add=True)
    plsc.subcore_barrier()

    pltpu.sync_copy(acc.at[pl.ds(sid * RPT, RPT)], out_hbm.at[cid, pl.ds(sid * RPT, RPT)])

    @pl.when(sid == 0)
    def _():
        pltpu.sync_copy(acc.at[pl.ds(TAIL_OFF, TAIL)], out_hbm.at[cid, pl.ds(TAIL_OFF, TAIL)])


BR = 2000


def _z_body(x_ref, w1_ref, invs_ref, z_ref):
    iso = invs_ref[0]
    y = jnp.dot(x_ref[...], w1_ref[...], preferred_element_type=jnp.float32)
    z_ref[...] = y * iso


def _z_call(x, W1, invs):
    return pl.pallas_call(
        _z_body,
        grid=(N // BR,),
        in_specs=[
            pl.BlockSpec((BR, F), lambda i: (i, 0)),
            pl.BlockSpec((F, H), lambda i: (0, 0)),
            pl.BlockSpec((2, BR, 1), lambda i: (0, i, 0)),
        ],
        out_specs=pl.BlockSpec((BR, H), lambda i: (i, 0)),
        out_shape=jax.ShapeDtypeStruct((N, H), jnp.float32),
    )(x, W1, invs)


def _head_body(p_ref, invs_ref, x_ref, w2_ref, b1_ref, w3_ref, b3_ref, o_ref):
    agg = p_ref[0] + p_ref[1]
    isi = invs_ref[1]
    h = agg * isi
    h = h + jnp.dot(x_ref[...], w2_ref[...], preferred_element_type=jnp.float32)
    h = h + b1_ref[...]
    h = jnp.where(h > 0, h, 0.2 * h)
    logits = jnp.dot(h, w3_ref[...], preferred_element_type=jnp.float32) + b3_ref[...]
    m = jnp.max(logits, axis=1, keepdims=True)
    ex = jnp.exp(logits - m)
    o_ref[...] = ex / jnp.sum(ex, axis=1, keepdims=True)


def _head_call(p, invs, x, W2, b1, W3, b3):
    return pl.pallas_call(
        _head_body,
        grid=(N // BR,),
        in_specs=[
            pl.BlockSpec((NC, BR, H), lambda i: (0, i, 0)),
            pl.BlockSpec((2, BR, 1), lambda i: (0, i, 0)),
            pl.BlockSpec((BR, F), lambda i: (i, 0)),
            pl.BlockSpec((F, H), lambda i: (0, 0)),
            pl.BlockSpec((1, H), lambda i: (0, 0)),
            pl.BlockSpec((H, L), lambda i: (0, 0)),
            pl.BlockSpec((1, L), lambda i: (0, 0)),
        ],
        out_specs=pl.BlockSpec((BR, L), lambda i: (i, 0)),
        out_shape=jax.ShapeDtypeStruct((N, L), jnp.float32),
    )(p, invs, x, W2, b1, W3, b3)


def kernel(x, edge_index, i, W1, W2, b1, W3, b3):
    del i
    src = jnp.asarray(edge_index[0], jnp.int32)
    dst = jnp.asarray(edge_index[1], jnp.int32)
    zeros_rows = jnp.zeros((N, H), jnp.float32)
    zeros_n = jnp.zeros((NP,), jnp.float32)
    ones_c = jnp.ones((CHUNK,), jnp.float32)
    src3 = src.reshape(NW, NCHUNK, CHUNK)
    dst3 = dst.reshape(NW, NCHUNK, CHUNK)

    degp = _deg_kernel(src3, dst3, zeros_n, ones_c)
    invs = _invs_call(degp)
    z = _z_call(x, W1, invs)
    p = _edge_kernel(z, src, dst3, zeros_rows)
    out = _head_call(p, invs, x, W2,
                     b1.reshape(1, H), W3, b3.reshape(1, L))
    return out


# speedup vs baseline: 32.8515x; 1.0284x over previous
import functools

import jax
import jax.numpy as jnp
from jax import lax
from jax.experimental import pallas as pl
from jax.experimental.pallas import tpu as pltpu
from jax.experimental.pallas import tpu_sc as plsc

N = 10000
E = 320000
F = 128
H = 128
L = 64

NC = 2
NS = 16
NW = NC * NS
EPW = E // NW
CHUNK = 80
NCHUNK = EPW // CHUNK
RPT = 624
TAIL_OFF = RPT * NS
TAIL = N - TAIL_OFF
NP = 10240

_mesh = plsc.VectorSubcoreMesh(core_axis_name="c", subcore_axis_name="s")


@functools.partial(
    pl.kernel,
    out_type=jax.ShapeDtypeStruct((2 * NC, 1, NP), jnp.float32),
    mesh=_mesh,
    scratch_types=[
        pltpu.VMEM((NCHUNK, CHUNK), jnp.int32),
        pltpu.VMEM((NCHUNK, CHUNK), jnp.int32),
        pltpu.VMEM((CHUNK,), jnp.float32),
        pltpu.VMEM_SHARED((NP,), jnp.float32),
        pltpu.VMEM_SHARED((NP,), jnp.float32),
    ],
)
def _deg_kernel(src3_hbm, dst3_hbm, zeros1_hbm, ones1_hbm, out_hbm,
                sidx_all, didx_all, ones_v, hs, hd):
    cid = lax.axis_index("c")
    sid = lax.axis_index("s")
    wid = cid * NS + sid

    pltpu.sync_copy(src3_hbm.at[wid], sidx_all)
    pltpu.sync_copy(dst3_hbm.at[wid], didx_all)
    pltpu.sync_copy(ones1_hbm, ones_v)

    @pl.when(sid == 0)
    def _():
        pltpu.sync_copy(zeros1_hbm, hs)
        pltpu.sync_copy(zeros1_hbm, hd)

    plsc.subcore_barrier()

    def body(g, carry):
        pltpu.sync_copy(ones_v, hs.at[sidx_all.at[g]], add=True)
        pltpu.sync_copy(ones_v, hd.at[didx_all.at[g]], add=True)
        return carry

    lax.fori_loop(0, NCHUNK, body, 0)
    plsc.subcore_barrier()

    pltpu.sync_copy(hs.at[pl.ds(sid * 640, 640)], out_hbm.at[cid * 2, 0, pl.ds(sid * 640, 640)])
    pltpu.sync_copy(hd.at[pl.ds(sid * 640, 640)], out_hbm.at[cid * 2 + 1, 0, pl.ds(sid * 640, 640)])


def _inv_sqrt_deg(degp_blk, pick_even):
    rows = lax.broadcasted_iota(jnp.int32, (2 * NC, 1), 0)
    sel = (rows % 2 == (0 if pick_even else 1)).astype(jnp.float32)
    cd = (((0,), (0,)), ((), ()))
    d = lax.dot_general(degp_blk, sel, cd, preferred_element_type=jnp.float32)
    return lax.rsqrt(jnp.maximum(d, 1.0))


CCH = 80
CNCH = EPW // CCH


@functools.partial(
    pl.kernel,
    out_type=jax.ShapeDtypeStruct((NC, NP, H), jnp.float32),
    mesh=_mesh,
    scratch_types=[
        pltpu.VMEM((EPW,), jnp.int32),
        pltpu.VMEM((CNCH, CCH), jnp.int32),
        pltpu.VMEM((CCH, H), jnp.float32),
        pltpu.VMEM((CCH, H), jnp.float32),
        pltpu.VMEM_SHARED((N, H), jnp.float32),
        pltpu.SemaphoreType.DMA,
        pltpu.SemaphoreType.DMA,
    ],
)
def _edge_kernel(z_hbm, src_hbm, dst3_hbm, zrows_hbm, out_hbm,
                 sidx, didx_all, rows0, rows1, acc, sem0, sem1):
    cid = lax.axis_index("c")
    sid = lax.axis_index("s")
    wid = cid * NS + sid

    pltpu.sync_copy(src_hbm.at[pl.ds(wid * EPW, EPW)], sidx)
    pltpu.sync_copy(dst3_hbm.at[wid], didx_all)

    pltpu.sync_copy(zrows_hbm.at[pl.ds(sid * RPT, RPT)], acc.at[pl.ds(sid * RPT, RPT)])

    @pl.when(sid == 0)
    def _():
        pltpu.sync_copy(zrows_hbm.at[pl.ds(TAIL_OFF, TAIL)], acc.at[pl.ds(TAIL_OFF, TAIL)])

    plsc.subcore_barrier()

    dummy0 = pltpu.make_async_copy(zrows_hbm.at[pl.ds(0, CCH)], rows0, sem0)
    dummy1 = pltpu.make_async_copy(zrows_hbm.at[pl.ds(0, CCH)], rows1, sem1)

    pltpu.async_copy(z_hbm.at[sidx.at[pl.ds(0, CCH)]], rows0, sem0)

    def body(g, carry):
        c0 = 2 * g
        c1 = 2 * g + 1
        pltpu.async_copy(z_hbm.at[sidx.at[pl.ds(c1 * CCH, CCH)]], rows1, sem1)
        dummy0.wait()
        pltpu.sync_copy(rows0, acc.at[didx_all.at[c0]], add=True)
        pltpu.async_copy(z_hbm.at[sidx.at[pl.ds((c0 + 2) * CCH, CCH)]], rows0, sem0)
        dummy1.wait()
        pltpu.sync_copy(rows1, acc.at[didx_all.at[c1]], add=True)
        return carry

    lax.fori_loop(0, CNCH // 2, body, 0)
    dummy0.wait()
    pltpu.sync_copy(rows0, acc.at[didx_all.at[CNCH - 1]], add=True)
    plsc.subcore_barrier()

    pltpu.sync_copy(acc.at[pl.ds(sid * RPT, RPT)], out_hbm.at[cid, pl.ds(sid * RPT, RPT)])

    @pl.when(sid == 0)
    def _():
        pltpu.sync_copy(acc.at[pl.ds(TAIL_OFF, TAIL)], out_hbm.at[cid, pl.ds(TAIL_OFF, TAIL)])


BR = 2048


def _z_body(x_ref, w1_ref, degp_ref, z_ref):
    iso = _inv_sqrt_deg(degp_ref[...][:, 0, :], True)
    y = jnp.dot(x_ref[...], w1_ref[...], preferred_element_type=jnp.float32)
    z_ref[...] = y * iso


def _z_call(x, W1, degp):
    return pl.pallas_call(
        _z_body,
        grid=(NP // BR,),
        in_specs=[
            pl.BlockSpec((BR, F), lambda i: (i, 0)),
            pl.BlockSpec((F, H), lambda i: (0, 0)),
            pl.BlockSpec((2 * NC, 1, BR), lambda i: (0, 0, i)),
        ],
        out_specs=pl.BlockSpec((BR, H), lambda i: (i, 0)),
        out_shape=jax.ShapeDtypeStruct((NP, H), jnp.float32),
    )(x, W1, degp)


def _head_body(p_ref, degp_ref, x_ref, w2_ref, b1_ref, w3_ref, b3_ref, o_ref):
    agg = p_ref[0] + p_ref[1]
    isi = _inv_sqrt_deg(degp_ref[...][:, 0, :], False)
    h = agg * isi
    h = h + jnp.dot(x_ref[...], w2_ref[...], preferred_element_type=jnp.float32)
    h = h + b1_ref[...]
    h = jnp.where(h > 0, h, 0.2 * h)
    logits = jnp.dot(h, w3_ref[...], preferred_element_type=jnp.float32) + b3_ref[...]
    m = jnp.max(logits, axis=1, keepdims=True)
    ex = jnp.exp(logits - m)
    o_ref[...] = ex / jnp.sum(ex, axis=1, keepdims=True)


def _head_call(p, degp, x, W2, b1, W3, b3):
    return pl.pallas_call(
        _head_body,
        grid=(NP // BR,),
        in_specs=[
            pl.BlockSpec((NC, BR, H), lambda i: (0, i, 0)),
            pl.BlockSpec((2 * NC, 1, BR), lambda i: (0, 0, i)),
            pl.BlockSpec((BR, F), lambda i: (i, 0)),
            pl.BlockSpec((F, H), lambda i: (0, 0)),
            pl.BlockSpec((1, H), lambda i: (0, 0)),
            pl.BlockSpec((H, L), lambda i: (0, 0)),
            pl.BlockSpec((1, L), lambda i: (0, 0)),
        ],
        out_specs=pl.BlockSpec((BR, L), lambda i: (i, 0)),
        out_shape=jax.ShapeDtypeStruct((NP, L), jnp.float32),
    )(p, degp, x, W2, b1, W3, b3)


def kernel(x, edge_index, i, W1, W2, b1, W3, b3):
    del i
    src = jnp.asarray(edge_index[0], jnp.int32)
    dst = jnp.asarray(edge_index[1], jnp.int32)
    zeros_rows = jnp.zeros((N, H), jnp.float32)
    zeros_n = jnp.zeros((NP,), jnp.float32)
    ones_c = jnp.ones((CHUNK,), jnp.float32)
    src3 = src.reshape(NW, NCHUNK, CHUNK)
    dst3 = dst.reshape(NW, NCHUNK, CHUNK)

    x_pad = jnp.pad(x, ((0, NP - N), (0, 0)))

    degp = _deg_kernel(src3, dst3, zeros_n, ones_c)
    z = _z_call(x_pad, W1, degp)
    p = _edge_kernel(z, src, dst3, zeros_rows)
    out = _head_call(p, degp, x_pad, W2,
                     b1.reshape(1, H), W3, b3.reshape(1, L))
    return out[:N]
